# Initial kernel scaffold; baseline (speedup 1.0000x reference)
import jax
import jax.numpy as jnp
from jax.experimental import pallas as pl


def kernel(xyz1, xyz2, points1, points2, W0, b0, g0, be0, W1, b1, g1, be1):
    raise NotImplementedError("write your pallas kernel here")


# speedup vs baseline: 21.1606x; 21.1606x over previous
import jax
import jax.numpy as jnp
from jax import lax
from jax.experimental import pallas as pl


_NB = 512


def _pass_a_body(x1r, x2r, p1r, p2r, w0r, b0r, y0r, s0r, ss0r):
    nb = x1r.shape[2]
    s = x2r.shape[2]
    x1 = x1r[0]
    x2 = x2r[0]
    ones3 = jnp.ones((3, 1), jnp.float32)
    sq1 = lax.dot_general(x1 * x1, ones3, (((0,), (0,)), ((), ())),
                          precision=lax.Precision.HIGHEST)
    sq2 = jnp.sum(x2 * x2, axis=0, keepdims=True)
    cross = lax.dot_general(x1, x2, (((0,), (0,)), ((), ())))
    d = (-2.0 * cross + sq1) + sq2

    iota = lax.broadcasted_iota(jnp.int32, (nb, s), 1)
    work = d
    idxs = []
    vals = []
    for _ in range(3):
        vmin = jnp.min(work, axis=1, keepdims=True)
        hit = work == vmin
        ik = jnp.min(jnp.where(hit, iota, s), axis=1, keepdims=True)
        idxs.append(ik)
        vals.append(vmin)
        work = jnp.where(iota == ik, jnp.inf, work)

    r0 = 1.0 / (vals[0] + 1e-8)
    r1 = 1.0 / (vals[1] + 1e-8)
    r2 = 1.0 / (vals[2] + 1e-8)
    norm = r0 + r1 + r2
    w0 = r0 / norm
    w1 = r1 / norm
    w2 = r2 / norm

    zero = jnp.zeros((), jnp.float32)
    wmat = (jnp.where(iota == idxs[0], w0, zero)
            + jnp.where(iota == idxs[1], w1, zero)
            + jnp.where(iota == idxs[2], w2, zero))

    p2 = p2r[0]
    interp = lax.dot_general(p2, wmat, (((1,), (1,)), ((), ())))
    newp = interp + p1r[0]
    y0 = lax.dot_general(w0r[...], newp, (((1,), (0,)), ((), ()))) + b0r[...]
    y0r[...] = y0[None]
    s0r[...] = jnp.sum(y0, axis=1, keepdims=True)[None]
    ss0r[...] = jnp.sum(y0 * y0, axis=1, keepdims=True)[None]


def _pass_b_body(y0r, a0r, c0r, w1r, b1r, y1r, s1r, ss1r):
    h = jnp.maximum(a0r[...] * y0r[0] + c0r[...], 0.0)
    y1 = lax.dot_general(w1r[...], h, (((1,), (0,)), ((), ()))) + b1r[...]
    y1r[...] = y1[None]
    s1r[...] = jnp.sum(y1, axis=1, keepdims=True)[None]
    ss1r[...] = jnp.sum(y1 * y1, axis=1, keepdims=True)[None]


def _pass_c_body(y1r, a1r, c1r, outr):
    outr[...] = jnp.maximum(a1r[...] * y1r[0] + c1r[...], 0.0)[None]


def kernel(xyz1, xyz2, points1, points2, W0, b0, g0, be0, W1, b1, g1, be1):
    B, _, N = xyz1.shape
    S = xyz2.shape[2]
    D = points2.shape[1]
    C0 = W0.shape[0]
    C1 = W1.shape[0]
    NB = _NB
    NJ = N // NB
    M = float(B * N)

    b0c = b0.reshape(C0, 1)
    b1c = b1.reshape(C1, 1)

    y0, s0, ss0 = pl.pallas_call(
        _pass_a_body,
        grid=(B, NJ),
        in_specs=[
            pl.BlockSpec((1, 3, NB), lambda b, j: (b, 0, j)),
            pl.BlockSpec((1, 3, S), lambda b, j: (b, 0, 0)),
            pl.BlockSpec((1, D, NB), lambda b, j: (b, 0, j)),
            pl.BlockSpec((1, D, S), lambda b, j: (b, 0, 0)),
            pl.BlockSpec((C0, D), lambda b, j: (0, 0)),
            pl.BlockSpec((C0, 1), lambda b, j: (0, 0)),
        ],
        out_specs=[
            pl.BlockSpec((1, C0, NB), lambda b, j: (b, 0, j)),
            pl.BlockSpec((1, C0, 1), lambda b, j: (b * NJ + j, 0, 0)),
            pl.BlockSpec((1, C0, 1), lambda b, j: (b * NJ + j, 0, 0)),
        ],
        out_shape=[
            jax.ShapeDtypeStruct((B, C0, N), jnp.float32),
            jax.ShapeDtypeStruct((B * NJ, C0, 1), jnp.float32),
            jax.ShapeDtypeStruct((B * NJ, C0, 1), jnp.float32),
        ],
    )(xyz1, xyz2, points1, points2, W0, b0c)

    mean0 = jnp.sum(s0, axis=0) / M
    var0 = jnp.sum(ss0, axis=0) / M - mean0 * mean0
    a0 = g0.reshape(C0, 1) * lax.rsqrt(var0 + 1e-5)
    c0 = be0.reshape(C0, 1) - a0 * mean0

    y1, s1, ss1 = pl.pallas_call(
        _pass_b_body,
        grid=(B, NJ),
        in_specs=[
            pl.BlockSpec((1, C0, NB), lambda b, j: (b, 0, j)),
            pl.BlockSpec((C0, 1), lambda b, j: (0, 0)),
            pl.BlockSpec((C0, 1), lambda b, j: (0, 0)),
            pl.BlockSpec((C1, C0), lambda b, j: (0, 0)),
            pl.BlockSpec((C1, 1), lambda b, j: (0, 0)),
        ],
        out_specs=[
            pl.BlockSpec((1, C1, NB), lambda b, j: (b, 0, j)),
            pl.BlockSpec((1, C1, 1), lambda b, j: (b * NJ + j, 0, 0)),
            pl.BlockSpec((1, C1, 1), lambda b, j: (b * NJ + j, 0, 0)),
        ],
        out_shape=[
            jax.ShapeDtypeStruct((B, C1, N), jnp.float32),
            jax.ShapeDtypeStruct((B * NJ, C1, 1), jnp.float32),
            jax.ShapeDtypeStruct((B * NJ, C1, 1), jnp.float32),
        ],
    )(y0, a0, c0, W1, b1c)

    mean1 = jnp.sum(s1, axis=0) / M
    var1 = jnp.sum(ss1, axis=0) / M - mean1 * mean1
    a1 = g1.reshape(C1, 1) * lax.rsqrt(var1 + 1e-5)
    c1 = be1.reshape(C1, 1) - a1 * mean1

    out = pl.pallas_call(
        _pass_c_body,
        grid=(B, NJ),
        in_specs=[
            pl.BlockSpec((1, C1, NB), lambda b, j: (b, 0, j)),
            pl.BlockSpec((C1, 1), lambda b, j: (0, 0)),
            pl.BlockSpec((C1, 1), lambda b, j: (0, 0)),
        ],
        out_specs=pl.BlockSpec((1, C1, NB), lambda b, j: (b, 0, j)),
        out_shape=jax.ShapeDtypeStruct((B, C1, N), jnp.float32),
    )(y1, a1, c1)

    return out
